# baseline (speedup 1.0000x reference)
import functools

import jax
import jax.numpy as jnp
from jax import lax
from jax.experimental import pallas as pl
from jax.experimental.pallas import tpu as pltpu
from jax.experimental.pallas import tpu_sc as plsc

N_NODES = 10000
N_EDGES = 320000
D_NODE = 128
D_EDGE = 16

NC = 2
NS = 16
NW = NC * NS
CH = 128
NCH = 80
PER_W = CH * NCH
E_PAD = NW * PER_W
PAD_NODE = N_NODES
NODE_PAD = 10240
ROWS_PER_SUB = NODE_PAD // NS
ZCH = ROWS_PER_SUB // CH

D_HALF = D_NODE // 2

_SQRT_HALF = 0.7071067811865476


def _gelu(x):
    return 0.5 * x * (1.0 + lax.erf(x * _SQRT_HALF))


def _sc_gather_body(table_hbm, sidx_hbm, didx_hbm, srows_hbm, drows_hbm,
                    sidx_v, didx_v, bs0, bs1, bd0, bd1,
                    gs0, gs1, gd0, gd1, ws0, ws1, wd0, wd1):
    c = lax.axis_index("c")
    s = lax.axis_index("s")
    wid = s * NC + c
    pltpu.sync_copy(sidx_hbm.at[c, pl.ds(wid * NCH, NCH)], sidx_v)
    pltpu.sync_copy(didx_hbm.at[c, pl.ds(wid * NCH, NCH)], didx_v)
    bs, bd = (bs0, bs1), (bd0, bd1)
    gs, gd = (gs0, gs1), (gd0, gd1)
    ws, wd = (ws0, ws1), (wd0, wd1)

    def start_gather(j, p):
        pltpu.async_copy(table_hbm.at[sidx_v.at[j]], bs[p], gs[p])
        pltpu.async_copy(table_hbm.at[didx_v.at[j]], bd[p], gd[p])

    def wait_writes(j, p):
        base = wid * PER_W + j * CH
        pltpu.make_async_copy(bs[p], srows_hbm.at[pl.ds(base, CH)],
                              ws[p]).wait()
        pltpu.make_async_copy(bd[p], drows_hbm.at[pl.ds(base, CH)],
                              wd[p]).wait()

    start_gather(0, 0)

    def body(ii, carry):
        for p in (0, 1):
            j = 2 * ii + p
            np_ = 1 - p

            @pl.when(j + 1 < NCH)
            def _prefetch():
                @pl.when(j >= 1)
                def _drain():
                    wait_writes(j - 1, np_)
                start_gather(j + 1, np_)

            base = wid * PER_W + j * CH
            pltpu.make_async_copy(table_hbm.at[sidx_v.at[j]], bs[p],
                                  gs[p]).wait()
            pltpu.async_copy(bs[p], srows_hbm.at[pl.ds(base, CH)], ws[p])
            pltpu.make_async_copy(table_hbm.at[didx_v.at[j]], bd[p],
                                  gd[p]).wait()
            pltpu.async_copy(bd[p], drows_hbm.at[pl.ds(base, CH)], wd[p])
        return carry

    lax.fori_loop(0, NCH // 2, body, 0)
    wait_writes(NCH - 2, 0)
    wait_writes(NCH - 1, 1)


@functools.cache
def _get_sc_gather():
    return pl.kernel(
        _sc_gather_body,
        out_type=(
            jax.ShapeDtypeStruct((E_PAD, D_NODE), jnp.float32),
            jax.ShapeDtypeStruct((E_PAD, D_NODE), jnp.float32),
        ),
        mesh=plsc.VectorSubcoreMesh(core_axis_name="c", subcore_axis_name="s",
                                    num_cores=NC, num_subcores=NS),
        scratch_types=(
            [pltpu.VMEM((NCH, CH), jnp.int32)] * 2
            + [pltpu.VMEM((CH, D_NODE), jnp.float32)] * 4
            + [pltpu.SemaphoreType.DMA] * 8
        ),
    )


def _tc_msg_body(srows, drows, ev, w1a, w1b, w1c, b1, w2, b2, msg1, msg2):
    sr = srows[...].astype(jnp.bfloat16)
    dr = drows[...].astype(jnp.bfloat16)
    w1a_ = w1a[...].astype(jnp.bfloat16)
    w1b_ = w1b[...].astype(jnp.bfloat16)
    w2_ = w2[...].astype(jnp.bfloat16)
    e = jnp.dot(ev[...], w1c[...], preferred_element_type=jnp.float32) + b1[...]
    sa = jnp.dot(sr, w1a_, preferred_element_type=jnp.float32)
    sb = jnp.dot(sr, w1b_, preferred_element_type=jnp.float32)
    da = jnp.dot(dr, w1a_, preferred_element_type=jnp.float32)
    db = jnp.dot(dr, w1b_, preferred_element_type=jnp.float32)
    h1 = _gelu(sa + db + e).astype(jnp.bfloat16)
    h2 = _gelu(da + sb + e).astype(jnp.bfloat16)
    msg1[...] = _gelu(jnp.dot(h1, w2_, preferred_element_type=jnp.float32)
                      + b2[...])
    msg2[...] = _gelu(jnp.dot(h2, w2_, preferred_element_type=jnp.float32)
                      + b2[...])


def _tc_msgs(srows, drows, ev, W1, b1, W2, b2, blk=2048):
    nblk = E_PAD // blk
    w1a = W1[:D_NODE]
    w1b = W1[D_NODE:2 * D_NODE]
    w1c = W1[2 * D_NODE:]
    d_hid = W1.shape[1]
    full = lambda a: pl.BlockSpec(a.shape, lambda i: (0,) * a.ndim)
    return pl.pallas_call(
        _tc_msg_body,
        grid=(nblk,),
        in_specs=[
            pl.BlockSpec((blk, D_NODE), lambda i: (i, 0)),
            pl.BlockSpec((blk, D_NODE), lambda i: (i, 0)),
            pl.BlockSpec((blk, D_EDGE), lambda i: (i, 0)),
            full(w1a), full(w1b), full(w1c),
            pl.BlockSpec((1, d_hid), lambda i: (0, 0)),
            full(W2),
            pl.BlockSpec((1, D_NODE), lambda i: (0, 0)),
        ],
        out_specs=[
            pl.BlockSpec((blk, D_NODE), lambda i: (i, 0)),
            pl.BlockSpec((blk, D_NODE), lambda i: (i, 0)),
        ],
        out_shape=[
            jax.ShapeDtypeStruct((E_PAD, D_NODE), jnp.float32),
            jax.ShapeDtypeStruct((E_PAD, D_NODE), jnp.float32),
        ],
        compiler_params=pltpu.CompilerParams(
            dimension_semantics=("arbitrary",)),
    )(srows, drows, ev, w1a, w1b, w1c, b1.reshape(1, -1), W2,
      b2.reshape(1, -1))


def _sc_scatter_body(msg1_hbm, msg2_hbm, sidx_hbm, didx_hbm, zeros_hbm,
                     out_hbm, sidx_v, didx_v, mba, mbb, boxes_sh, rsa, rsb):
    c = lax.axis_index("c")
    s = lax.axis_index("s")
    wid = s * NC + c

    pltpu.sync_copy(zeros_hbm, mba)

    def zero_chunk(k, carry):
        pltpu.sync_copy(mba, boxes_sh.at[pl.ds(s * ROWS_PER_SUB + k * CH, CH)])
        return carry

    lax.fori_loop(0, ZCH, zero_chunk, 0)
    plsc.subcore_barrier()

    half = NCH // 2
    for h in (0, 1):
        pltpu.sync_copy(sidx_hbm.at[pl.ds(wid * NCH + h * half, half)],
                        sidx_v)
        pltpu.sync_copy(didx_hbm.at[pl.ds(wid * NCH + h * half, half)],
                        didx_v)
        h0 = wid * PER_W + h * half * CH
        pltpu.async_copy(msg1_hbm.at[pl.ds(h0, CH)], mba, rsa)

        def chunk(j, carry):
            base = h0 + j * CH
            pltpu.async_copy(msg2_hbm.at[pl.ds(base, CH)], mbb, rsb)
            pltpu.make_async_copy(msg1_hbm.at[pl.ds(base, CH)], mba,
                                  rsa).wait()
            pltpu.sync_copy(mba, boxes_sh.at[sidx_v.at[j]], add=True)

            @pl.when(j + 1 < half)
            def _prefetch():
                pltpu.async_copy(msg1_hbm.at[pl.ds(base + CH, CH)], mba, rsa)

            pltpu.make_async_copy(msg2_hbm.at[pl.ds(base, CH)], mbb,
                                  rsb).wait()
            pltpu.sync_copy(mbb, boxes_sh.at[didx_v.at[j]], add=True)
            return carry

        lax.fori_loop(0, half, chunk, 0)
    plsc.subcore_barrier()

    def dump_chunk(k, carry):
        r = s * ROWS_PER_SUB + k * CH
        pltpu.sync_copy(boxes_sh.at[pl.ds(r, CH)], mba)
        pltpu.sync_copy(mba, out_hbm.at[c, pl.ds(r, CH)])
        return carry

    lax.fori_loop(0, ZCH, dump_chunk, 0)


@functools.cache
def _get_sc_scatter():
    return pl.kernel(
        _sc_scatter_body,
        out_type=jax.ShapeDtypeStruct((NC, NODE_PAD, D_NODE), jnp.float32),
        mesh=plsc.VectorSubcoreMesh(core_axis_name="c", subcore_axis_name="s",
                                    num_cores=NC, num_subcores=NS),
        scratch_types=(
            [pltpu.VMEM((NCH // 2, CH), jnp.int32)] * 2
            + [pltpu.VMEM((CH, D_NODE), jnp.float32)] * 2
            + [pltpu.VMEM_SHARED((NODE_PAD, D_NODE), jnp.float32)]
            + [pltpu.SemaphoreType.DMA] * 2
        ),
    )


def _tc_combine_body(nv, box0, box1, w3a, w3b, b3, w4, b4, out):
    boxes = box0[...] + box1[...]
    h = _gelu(jnp.dot(nv[...], w3a[...], preferred_element_type=jnp.float32)
              + jnp.dot(boxes, w3b[...], preferred_element_type=jnp.float32)
              + b3[...])
    out[...] = _gelu(jnp.dot(h, w4[...], preferred_element_type=jnp.float32)
                     + b4[...])


def _tc_combine(nv, box0, box1, W3, b3, W4, b4, blk=2000):
    nblk = N_NODES // blk
    w3a = W3[:D_NODE]
    w3b = W3[D_NODE:]
    d_hid = W3.shape[1]
    full = lambda a: pl.BlockSpec(a.shape, lambda i: (0,) * a.ndim)
    return pl.pallas_call(
        _tc_combine_body,
        grid=(nblk,),
        in_specs=[
            pl.BlockSpec((blk, D_NODE), lambda i: (i, 0)),
            pl.BlockSpec((blk, D_NODE), lambda i: (i, 0)),
            pl.BlockSpec((blk, D_NODE), lambda i: (i, 0)),
            full(w3a), full(w3b),
            pl.BlockSpec((1, d_hid), lambda i: (0, 0)),
            full(W4),
            pl.BlockSpec((1, D_NODE), lambda i: (0, 0)),
        ],
        out_specs=pl.BlockSpec((blk, D_NODE), lambda i: (i, 0)),
        out_shape=jax.ShapeDtypeStruct((N_NODES, D_NODE), jnp.float32),
        compiler_params=pltpu.CompilerParams(
            dimension_semantics=("arbitrary",)),
    )(nv, box0, box1, w3a, w3b, b3.reshape(1, -1), W4, b4.reshape(1, -1))


def kernel(node_vectors, edge_vectors, edge_indices, W1, b1, W2, b2, W3, b3,
           W4, b4):
    table = jnp.zeros((NODE_PAD, D_NODE), jnp.float32)
    table = table.at[:N_NODES].set(node_vectors)
    table = jnp.concatenate([table, table], axis=0)
    npad = E_PAD - N_EDGES
    src = jnp.concatenate(
        [edge_indices[:, 0], jnp.full((npad,), PAD_NODE, jnp.int32)])
    dst = jnp.concatenate(
        [edge_indices[:, 1], jnp.full((npad,), PAD_NODE, jnp.int32)])
    sidx = src.reshape(NW * NCH, CH)
    didx = dst.reshape(NW * NCH, CH)
    sidx2 = jnp.stack([sidx, sidx + NODE_PAD])
    didx2 = jnp.stack([didx, didx + NODE_PAD])
    ev = jnp.concatenate(
        [edge_vectors, jnp.zeros((npad, D_EDGE), jnp.float32)])

    srows, drows = _get_sc_gather()(table, sidx2, didx2)
    msg1, msg2 = _tc_msgs(srows, drows, ev, W1, b1, W2, b2)
    zeros_blk = jnp.zeros((CH, D_NODE), jnp.float32)
    partials = _get_sc_scatter()(msg1, msg2, sidx, didx, zeros_blk)
    return _tc_combine(node_vectors, partials[0, :N_NODES],
                       partials[1, :N_NODES], W3, b3, W4, b4)


# speedup vs baseline: 1.1229x; 1.1229x over previous
import functools

import jax
import jax.numpy as jnp
from jax import lax
from jax.experimental import pallas as pl
from jax.experimental.pallas import tpu as pltpu
from jax.experimental.pallas import tpu_sc as plsc

N_NODES = 10000
N_EDGES = 320000
D_NODE = 128
D_EDGE = 16

NC = 2
NS = 16
NW = NC * NS
CH = 128
NCH = 80
PER_W = CH * NCH
E_PAD = NW * PER_W
N_HALF = 2
NCH_C = NCH // N_HALF
PER_WC = CH * NCH_C
E_C = NW * PER_WC
PAD_NODE = N_NODES
NODE_PAD = 10240
ROWS_PER_SUB = NODE_PAD // NS
ZCH = ROWS_PER_SUB // CH

_SQRT_HALF = 0.7071067811865476


def _gelu(x):
    return 0.5 * x * (1.0 + lax.erf(x * _SQRT_HALF))


def _sc_gather_body(table_hbm, sidx_hbm, didx_hbm, srows_hbm, drows_hbm,
                    sidx_v, didx_v, bs0, bs1, bd0, bd1,
                    gs0, gs1, gd0, gd1, ws0, ws1, wd0, wd1):
    c = lax.axis_index("c")
    s = lax.axis_index("s")
    wid = s * NC + c
    pltpu.sync_copy(sidx_hbm.at[pl.ds(wid * NCH_C, NCH_C)], sidx_v)
    pltpu.sync_copy(didx_hbm.at[pl.ds(wid * NCH_C, NCH_C)], didx_v)
    bs, bd = (bs0, bs1), (bd0, bd1)
    gs, gd = (gs0, gs1), (gd0, gd1)
    ws, wd = (ws0, ws1), (wd0, wd1)

    def start_gather(j, p):
        pltpu.async_copy(table_hbm.at[sidx_v.at[j]], bs[p], gs[p])
        pltpu.async_copy(table_hbm.at[didx_v.at[j]], bd[p], gd[p])

    def wait_writes(j, p):
        base = wid * PER_WC + j * CH
        pltpu.make_async_copy(bs[p], srows_hbm.at[pl.ds(base, CH)],
                              ws[p]).wait()
        pltpu.make_async_copy(bd[p], drows_hbm.at[pl.ds(base, CH)],
                              wd[p]).wait()

    start_gather(0, 0)

    def body(ii, carry):
        for p in (0, 1):
            j = 2 * ii + p
            np_ = 1 - p

            @pl.when(j + 1 < NCH_C)
            def _prefetch():
                @pl.when(j >= 1)
                def _drain():
                    wait_writes(j - 1, np_)
                start_gather(j + 1, np_)

            base = wid * PER_WC + j * CH
            pltpu.make_async_copy(table_hbm.at[sidx_v.at[j]], bs[p],
                                  gs[p]).wait()
            pltpu.async_copy(bs[p], srows_hbm.at[pl.ds(base, CH)], ws[p])
            pltpu.make_async_copy(table_hbm.at[didx_v.at[j]], bd[p],
                                  gd[p]).wait()
            pltpu.async_copy(bd[p], drows_hbm.at[pl.ds(base, CH)], wd[p])
        return carry

    lax.fori_loop(0, NCH_C // 2, body, 0)
    wait_writes(NCH_C - 2, 0)
    wait_writes(NCH_C - 1, 1)


@functools.cache
def _get_sc_gather():
    return pl.kernel(
        _sc_gather_body,
        out_type=(
            jax.ShapeDtypeStruct((E_C, D_NODE), jnp.float32),
            jax.ShapeDtypeStruct((E_C, D_NODE), jnp.float32),
        ),
        mesh=plsc.VectorSubcoreMesh(core_axis_name="c", subcore_axis_name="s",
                                    num_cores=NC, num_subcores=NS),
        scratch_types=(
            [pltpu.VMEM((NCH_C, CH), jnp.int32)] * 2
            + [pltpu.VMEM((CH, D_NODE), jnp.float32)] * 4
            + [pltpu.SemaphoreType.DMA] * 8
        ),
    )


def _tc_msg_body(srows, drows, ev, w1a, w1b, w1c, b1, w2, b2, msg1, msg2):
    sr = srows[...].astype(jnp.bfloat16)
    dr = drows[...].astype(jnp.bfloat16)
    w1a_ = w1a[...].astype(jnp.bfloat16)
    w1b_ = w1b[...].astype(jnp.bfloat16)
    w2_ = w2[...].astype(jnp.bfloat16)
    e = jnp.dot(ev[...], w1c[...], preferred_element_type=jnp.float32) + b1[...]
    sa = jnp.dot(sr, w1a_, preferred_element_type=jnp.float32)
    sb = jnp.dot(sr, w1b_, preferred_element_type=jnp.float32)
    da = jnp.dot(dr, w1a_, preferred_element_type=jnp.float32)
    db = jnp.dot(dr, w1b_, preferred_element_type=jnp.float32)
    h1 = _gelu(sa + db + e).astype(jnp.bfloat16)
    h2 = _gelu(da + sb + e).astype(jnp.bfloat16)
    msg1[...] = _gelu(jnp.dot(h1, w2_, preferred_element_type=jnp.float32)
                      + b2[...])
    msg2[...] = _gelu(jnp.dot(h2, w2_, preferred_element_type=jnp.float32)
                      + b2[...])


def _tc_msgs(srows, drows, ev, W1, b1, W2, b2, blk=2048):
    nblk = E_C // blk
    w1a = W1[:D_NODE]
    w1b = W1[D_NODE:2 * D_NODE]
    w1c = W1[2 * D_NODE:]
    d_hid = W1.shape[1]
    full = lambda a: pl.BlockSpec(a.shape, lambda i: (0,) * a.ndim)
    return pl.pallas_call(
        _tc_msg_body,
        grid=(nblk,),
        in_specs=[
            pl.BlockSpec((blk, D_NODE), lambda i: (i, 0)),
            pl.BlockSpec((blk, D_NODE), lambda i: (i, 0)),
            pl.BlockSpec((blk, D_EDGE), lambda i: (i, 0)),
            full(w1a), full(w1b), full(w1c),
            pl.BlockSpec((1, d_hid), lambda i: (0, 0)),
            full(W2),
            pl.BlockSpec((1, D_NODE), lambda i: (0, 0)),
        ],
        out_specs=[
            pl.BlockSpec((blk, D_NODE), lambda i: (i, 0)),
            pl.BlockSpec((blk, D_NODE), lambda i: (i, 0)),
        ],
        out_shape=[
            jax.ShapeDtypeStruct((E_C, D_NODE), jnp.float32),
            jax.ShapeDtypeStruct((E_C, D_NODE), jnp.float32),
        ],
        compiler_params=pltpu.CompilerParams(
            dimension_semantics=("arbitrary",)),
    )(srows, drows, ev, w1a, w1b, w1c, b1.reshape(1, -1), W2,
      b2.reshape(1, -1))


def _sc_scatter_body(msg1_hbm, msg2_hbm, sidx_hbm, didx_hbm, zeros_hbm,
                     out_hbm, sidx_v, didx_v, mba, mbb, boxes_sh, rsa, rsb):
    c = lax.axis_index("c")
    s = lax.axis_index("s")
    wid = s * NC + c

    pltpu.sync_copy(zeros_hbm, mba)

    def zero_chunk(k, carry):
        pltpu.sync_copy(mba, boxes_sh.at[pl.ds(s * ROWS_PER_SUB + k * CH, CH)])
        return carry

    lax.fori_loop(0, ZCH, zero_chunk, 0)
    plsc.subcore_barrier()

    pltpu.sync_copy(sidx_hbm.at[pl.ds(wid * NCH_C, NCH_C)], sidx_v)
    pltpu.sync_copy(didx_hbm.at[pl.ds(wid * NCH_C, NCH_C)], didx_v)

    base0 = wid * PER_WC
    pltpu.async_copy(msg1_hbm.at[pl.ds(base0, CH)], mba, rsa)

    def chunk(j, carry):
        base = base0 + j * CH
        pltpu.async_copy(msg2_hbm.at[pl.ds(base, CH)], mbb, rsb)
        pltpu.make_async_copy(msg1_hbm.at[pl.ds(base, CH)], mba, rsa).wait()
        pltpu.sync_copy(mba, boxes_sh.at[sidx_v.at[j]], add=True)

        @pl.when(j + 1 < NCH_C)
        def _prefetch():
            pltpu.async_copy(msg1_hbm.at[pl.ds(base + CH, CH)], mba, rsa)

        pltpu.make_async_copy(msg2_hbm.at[pl.ds(base, CH)], mbb, rsb).wait()
        pltpu.sync_copy(mbb, boxes_sh.at[didx_v.at[j]], add=True)
        return carry

    lax.fori_loop(0, NCH_C, chunk, 0)
    plsc.subcore_barrier()

    def dump_chunk(k, carry):
        r = s * ROWS_PER_SUB + k * CH
        pltpu.sync_copy(boxes_sh.at[pl.ds(r, CH)], mba)
        pltpu.sync_copy(mba, out_hbm.at[c, pl.ds(r, CH)])
        return carry

    lax.fori_loop(0, ZCH, dump_chunk, 0)


@functools.cache
def _get_sc_scatter():
    return pl.kernel(
        _sc_scatter_body,
        out_type=jax.ShapeDtypeStruct((NC, NODE_PAD, D_NODE), jnp.float32),
        mesh=plsc.VectorSubcoreMesh(core_axis_name="c", subcore_axis_name="s",
                                    num_cores=NC, num_subcores=NS),
        scratch_types=(
            [pltpu.VMEM((NCH_C, CH), jnp.int32)] * 2
            + [pltpu.VMEM((CH, D_NODE), jnp.float32)] * 2
            + [pltpu.VMEM_SHARED((NODE_PAD, D_NODE), jnp.float32)]
            + [pltpu.SemaphoreType.DMA] * 2
        ),
    )


def _tc_combine_body(nv, b00, b01, b10, b11, w3a, w3b, b3, w4, b4, out):
    boxes = (b00[...] + b01[...]) + (b10[...] + b11[...])
    h = _gelu(jnp.dot(nv[...], w3a[...], preferred_element_type=jnp.float32)
              + jnp.dot(boxes, w3b[...], preferred_element_type=jnp.float32)
              + b3[...])
    out[...] = _gelu(jnp.dot(h, w4[...], preferred_element_type=jnp.float32)
                     + b4[...])


def _tc_combine(nv, boxes4, W3, b3, W4, b4, blk=2000):
    nblk = N_NODES // blk
    w3a = W3[:D_NODE]
    w3b = W3[D_NODE:]
    d_hid = W3.shape[1]
    full = lambda a: pl.BlockSpec(a.shape, lambda i: (0,) * a.ndim)
    row_spec = pl.BlockSpec((blk, D_NODE), lambda i: (i, 0))
    return pl.pallas_call(
        _tc_combine_body,
        grid=(nblk,),
        in_specs=[
            row_spec, row_spec, row_spec, row_spec, row_spec,
            full(w3a), full(w3b),
            pl.BlockSpec((1, d_hid), lambda i: (0, 0)),
            full(W4),
            pl.BlockSpec((1, D_NODE), lambda i: (0, 0)),
        ],
        out_specs=pl.BlockSpec((blk, D_NODE), lambda i: (i, 0)),
        out_shape=jax.ShapeDtypeStruct((N_NODES, D_NODE), jnp.float32),
        compiler_params=pltpu.CompilerParams(
            dimension_semantics=("arbitrary",)),
    )(nv, *boxes4, w3a, w3b, b3.reshape(1, -1), W4, b4.reshape(1, -1))


def kernel(node_vectors, edge_vectors, edge_indices, W1, b1, W2, b2, W3, b3,
           W4, b4):
    table = jnp.zeros((NODE_PAD, D_NODE), jnp.float32)
    table = table.at[:N_NODES].set(node_vectors)
    npad = E_PAD - N_EDGES
    src = jnp.concatenate(
        [edge_indices[:, 0], jnp.full((npad,), PAD_NODE, jnp.int32)])
    dst = jnp.concatenate(
        [edge_indices[:, 1], jnp.full((npad,), PAD_NODE, jnp.int32)])
    sidx = src.reshape(NW * NCH, CH)
    didx = dst.reshape(NW * NCH, CH)
    ev = jnp.concatenate(
        [edge_vectors, jnp.zeros((npad, D_EDGE), jnp.float32)])
    zeros_blk = jnp.zeros((CH, D_NODE), jnp.float32)

    gather = _get_sc_gather()
    scatter = _get_sc_scatter()
    boxes4 = []
    for h in range(N_HALF):
        rows = slice(h * NW * NCH_C, (h + 1) * NW * NCH_C)
        erows = slice(h * E_C, (h + 1) * E_C)
        sidx_h, didx_h = sidx[rows], didx[rows]
        srows, drows = gather(table, sidx_h, didx_h)
        msg1, msg2 = _tc_msgs(srows, drows, ev[erows], W1, b1, W2, b2)
        partials = scatter(msg1, msg2, sidx_h, didx_h, zeros_blk)
        boxes4.extend([partials[0, :N_NODES], partials[1, :N_NODES]])
    return _tc_combine(node_vectors, boxes4, W3, b3, W4, b4)


# speedup vs baseline: 1.1266x; 1.0033x over previous
import functools

import jax
import jax.numpy as jnp
from jax import lax
from jax.experimental import pallas as pl
from jax.experimental.pallas import tpu as pltpu
from jax.experimental.pallas import tpu_sc as plsc

N_NODES = 10000
N_EDGES = 320000
D_NODE = 128
D_EDGE = 16

NC = 2
NS = 16
NW = NC * NS
CH = 128
NCH = 80
PER_W = CH * NCH
E_PAD = NW * PER_W
N_HALF = 5
NCH_C = NCH // N_HALF
PER_WC = CH * NCH_C
E_C = NW * PER_WC
PAD_NODE = N_NODES
NODE_PAD = 10240
ROWS_PER_SUB = NODE_PAD // NS
ZCH = ROWS_PER_SUB // CH

_SQRT_HALF = 0.7071067811865476


def _gelu(x):
    return 0.5 * x * (1.0 + lax.erf(x * _SQRT_HALF))


def _sc_gather_body(table_hbm, sidx_hbm, didx_hbm, srows_hbm, drows_hbm,
                    sidx_v, didx_v, bs0, bs1, bd0, bd1,
                    gs0, gs1, gd0, gd1, ws0, ws1, wd0, wd1):
    c = lax.axis_index("c")
    s = lax.axis_index("s")
    wid = s * NC + c
    pltpu.sync_copy(sidx_hbm.at[pl.ds(wid * NCH_C, NCH_C)], sidx_v)
    pltpu.sync_copy(didx_hbm.at[pl.ds(wid * NCH_C, NCH_C)], didx_v)
    bs, bd = (bs0, bs1), (bd0, bd1)
    gs, gd = (gs0, gs1), (gd0, gd1)
    ws, wd = (ws0, ws1), (wd0, wd1)

    def start_gather(j, p):
        pltpu.async_copy(table_hbm.at[sidx_v.at[j]], bs[p], gs[p])
        pltpu.async_copy(table_hbm.at[didx_v.at[j]], bd[p], gd[p])

    def wait_writes(j, p):
        base = wid * PER_WC + j * CH
        pltpu.make_async_copy(bs[p], srows_hbm.at[pl.ds(base, CH)],
                              ws[p]).wait()
        pltpu.make_async_copy(bd[p], drows_hbm.at[pl.ds(base, CH)],
                              wd[p]).wait()

    start_gather(0, 0)

    def body(ii, carry):
        for p in (0, 1):
            j = 2 * ii + p
            np_ = 1 - p

            @pl.when(j + 1 < NCH_C)
            def _prefetch():
                @pl.when(j >= 1)
                def _drain():
                    wait_writes(j - 1, np_)
                start_gather(j + 1, np_)

            base = wid * PER_WC + j * CH
            pltpu.make_async_copy(table_hbm.at[sidx_v.at[j]], bs[p],
                                  gs[p]).wait()
            pltpu.async_copy(bs[p], srows_hbm.at[pl.ds(base, CH)], ws[p])
            pltpu.make_async_copy(table_hbm.at[didx_v.at[j]], bd[p],
                                  gd[p]).wait()
            pltpu.async_copy(bd[p], drows_hbm.at[pl.ds(base, CH)], wd[p])
        return carry

    lax.fori_loop(0, NCH_C // 2, body, 0)
    wait_writes(NCH_C - 2, 0)
    wait_writes(NCH_C - 1, 1)


@functools.cache
def _get_sc_gather():
    return pl.kernel(
        _sc_gather_body,
        out_type=(
            jax.ShapeDtypeStruct((E_C, D_NODE), jnp.float32),
            jax.ShapeDtypeStruct((E_C, D_NODE), jnp.float32),
        ),
        mesh=plsc.VectorSubcoreMesh(core_axis_name="c", subcore_axis_name="s",
                                    num_cores=NC, num_subcores=NS),
        scratch_types=(
            [pltpu.VMEM((NCH_C, CH), jnp.int32)] * 2
            + [pltpu.VMEM((CH, D_NODE), jnp.float32)] * 4
            + [pltpu.SemaphoreType.DMA] * 8
        ),
    )


def _tc_msg_body(srows, drows, ev, w1a, w1b, w1c, b1, w2, b2, msg1, msg2):
    sr = srows[...].astype(jnp.bfloat16)
    dr = drows[...].astype(jnp.bfloat16)
    w1a_ = w1a[...].astype(jnp.bfloat16)
    w1b_ = w1b[...].astype(jnp.bfloat16)
    w2_ = w2[...].astype(jnp.bfloat16)
    e = jnp.dot(ev[...], w1c[...], preferred_element_type=jnp.float32) + b1[...]
    sa = jnp.dot(sr, w1a_, preferred_element_type=jnp.float32)
    sb = jnp.dot(sr, w1b_, preferred_element_type=jnp.float32)
    da = jnp.dot(dr, w1a_, preferred_element_type=jnp.float32)
    db = jnp.dot(dr, w1b_, preferred_element_type=jnp.float32)
    h1 = _gelu(sa + db + e).astype(jnp.bfloat16)
    h2 = _gelu(da + sb + e).astype(jnp.bfloat16)
    msg1[...] = _gelu(jnp.dot(h1, w2_, preferred_element_type=jnp.float32)
                      + b2[...])
    msg2[...] = _gelu(jnp.dot(h2, w2_, preferred_element_type=jnp.float32)
                      + b2[...])


def _tc_msgs(srows, drows, ev, W1, b1, W2, b2, blk=2048):
    nblk = E_C // blk
    w1a = W1[:D_NODE]
    w1b = W1[D_NODE:2 * D_NODE]
    w1c = W1[2 * D_NODE:]
    d_hid = W1.shape[1]
    full = lambda a: pl.BlockSpec(a.shape, lambda i: (0,) * a.ndim)
    return pl.pallas_call(
        _tc_msg_body,
        grid=(nblk,),
        in_specs=[
            pl.BlockSpec((blk, D_NODE), lambda i: (i, 0)),
            pl.BlockSpec((blk, D_NODE), lambda i: (i, 0)),
            pl.BlockSpec((blk, D_EDGE), lambda i: (i, 0)),
            full(w1a), full(w1b), full(w1c),
            pl.BlockSpec((1, d_hid), lambda i: (0, 0)),
            full(W2),
            pl.BlockSpec((1, D_NODE), lambda i: (0, 0)),
        ],
        out_specs=[
            pl.BlockSpec((blk, D_NODE), lambda i: (i, 0)),
            pl.BlockSpec((blk, D_NODE), lambda i: (i, 0)),
        ],
        out_shape=[
            jax.ShapeDtypeStruct((E_C, D_NODE), jnp.float32),
            jax.ShapeDtypeStruct((E_C, D_NODE), jnp.float32),
        ],
        compiler_params=pltpu.CompilerParams(
            dimension_semantics=("arbitrary",)),
    )(srows, drows, ev, w1a, w1b, w1c, b1.reshape(1, -1), W2,
      b2.reshape(1, -1))


def _sc_scatter_body(msg1_hbm, msg2_hbm, sidx_hbm, didx_hbm, zeros_hbm,
                     out_hbm, sidx_v, didx_v, mba, mbb, boxes_sh, rsa, rsb):
    c = lax.axis_index("c")
    s = lax.axis_index("s")
    wid = s * NC + c

    pltpu.sync_copy(zeros_hbm, mba)

    def zero_chunk(k, carry):
        pltpu.sync_copy(mba, boxes_sh.at[pl.ds(s * ROWS_PER_SUB + k * CH, CH)])
        return carry

    lax.fori_loop(0, ZCH, zero_chunk, 0)
    plsc.subcore_barrier()

    pltpu.sync_copy(sidx_hbm.at[pl.ds(wid * NCH_C, NCH_C)], sidx_v)
    pltpu.sync_copy(didx_hbm.at[pl.ds(wid * NCH_C, NCH_C)], didx_v)

    base0 = wid * PER_WC
    pltpu.async_copy(msg1_hbm.at[pl.ds(base0, CH)], mba, rsa)

    def chunk(j, carry):
        base = base0 + j * CH
        pltpu.async_copy(msg2_hbm.at[pl.ds(base, CH)], mbb, rsb)
        pltpu.make_async_copy(msg1_hbm.at[pl.ds(base, CH)], mba, rsa).wait()
        pltpu.sync_copy(mba, boxes_sh.at[sidx_v.at[j]], add=True)

        @pl.when(j + 1 < NCH_C)
        def _prefetch():
            pltpu.async_copy(msg1_hbm.at[pl.ds(base + CH, CH)], mba, rsa)

        pltpu.make_async_copy(msg2_hbm.at[pl.ds(base, CH)], mbb, rsb).wait()
        pltpu.sync_copy(mbb, boxes_sh.at[didx_v.at[j]], add=True)
        return carry

    lax.fori_loop(0, NCH_C, chunk, 0)
    plsc.subcore_barrier()

    def dump_chunk(k, carry):
        r = s * ROWS_PER_SUB + k * CH
        pltpu.sync_copy(boxes_sh.at[pl.ds(r, CH)], mba)
        pltpu.sync_copy(mba, out_hbm.at[c, pl.ds(r, CH)])
        return carry

    lax.fori_loop(0, ZCH, dump_chunk, 0)


@functools.cache
def _get_sc_scatter():
    return pl.kernel(
        _sc_scatter_body,
        out_type=jax.ShapeDtypeStruct((NC, NODE_PAD, D_NODE), jnp.float32),
        mesh=plsc.VectorSubcoreMesh(core_axis_name="c", subcore_axis_name="s",
                                    num_cores=NC, num_subcores=NS),
        scratch_types=(
            [pltpu.VMEM((NCH_C, CH), jnp.int32)] * 2
            + [pltpu.VMEM((CH, D_NODE), jnp.float32)] * 2
            + [pltpu.VMEM_SHARED((NODE_PAD, D_NODE), jnp.float32)]
            + [pltpu.SemaphoreType.DMA] * 2
        ),
    )


def _tc_combine(nv, boxes, W3, b3, W4, b4, blk=2000):
    nblk = N_NODES // blk
    nb = len(boxes)
    w3a = W3[:D_NODE]
    w3b = W3[D_NODE:]
    d_hid = W3.shape[1]

    def body(*refs):
        nv_r = refs[0]
        box_rs = refs[1:1 + nb]
        w3a_r, w3b_r, b3_r, w4_r, b4_r, out = refs[1 + nb:]
        acc = box_rs[0][...]
        for r in box_rs[1:]:
            acc = acc + r[...]
        h = _gelu(
            jnp.dot(nv_r[...], w3a_r[...], preferred_element_type=jnp.float32)
            + jnp.dot(acc, w3b_r[...], preferred_element_type=jnp.float32)
            + b3_r[...])
        out[...] = _gelu(
            jnp.dot(h, w4_r[...], preferred_element_type=jnp.float32)
            + b4_r[...])

    full = lambda a: pl.BlockSpec(a.shape, lambda i: (0,) * a.ndim)
    row_spec = pl.BlockSpec((blk, D_NODE), lambda i: (i, 0))
    return pl.pallas_call(
        body,
        grid=(nblk,),
        in_specs=[row_spec] * (1 + nb) + [
            full(w3a), full(w3b),
            pl.BlockSpec((1, d_hid), lambda i: (0, 0)),
            full(W4),
            pl.BlockSpec((1, D_NODE), lambda i: (0, 0)),
        ],
        out_specs=pl.BlockSpec((blk, D_NODE), lambda i: (i, 0)),
        out_shape=jax.ShapeDtypeStruct((N_NODES, D_NODE), jnp.float32),
        compiler_params=pltpu.CompilerParams(
            dimension_semantics=("arbitrary",)),
    )(nv, *boxes, w3a, w3b, b3.reshape(1, -1), W4, b4.reshape(1, -1))


def kernel(node_vectors, edge_vectors, edge_indices, W1, b1, W2, b2, W3, b3,
           W4, b4):
    table = jnp.zeros((NODE_PAD, D_NODE), jnp.float32)
    table = table.at[:N_NODES].set(node_vectors)
    npad = E_PAD - N_EDGES
    src = jnp.concatenate(
        [edge_indices[:, 0], jnp.full((npad,), PAD_NODE, jnp.int32)])
    dst = jnp.concatenate(
        [edge_indices[:, 1], jnp.full((npad,), PAD_NODE, jnp.int32)])
    sidx = src.reshape(NW * NCH, CH)
    didx = dst.reshape(NW * NCH, CH)
    ev = jnp.concatenate(
        [edge_vectors, jnp.zeros((npad, D_EDGE), jnp.float32)])
    zeros_blk = jnp.zeros((CH, D_NODE), jnp.float32)

    gather = _get_sc_gather()
    scatter = _get_sc_scatter()
    boxes4 = []
    for h in range(N_HALF):
        rows = slice(h * NW * NCH_C, (h + 1) * NW * NCH_C)
        erows = slice(h * E_C, (h + 1) * E_C)
        sidx_h, didx_h = sidx[rows], didx[rows]
        srows, drows = gather(table, sidx_h, didx_h)
        msg1, msg2 = _tc_msgs(srows, drows, ev[erows], W1, b1, W2, b2)
        partials = scatter(msg1, msg2, sidx_h, didx_h, zeros_blk)
        boxes4.extend([partials[0, :N_NODES], partials[1, :N_NODES]])
    return _tc_combine(node_vectors, boxes4, W3, b3, W4, b4)


# speedup vs baseline: 1.1322x; 1.0050x over previous
import functools

import jax
import jax.numpy as jnp
from jax import lax
from jax.experimental import pallas as pl
from jax.experimental.pallas import tpu as pltpu
from jax.experimental.pallas import tpu_sc as plsc

N_NODES = 10000
N_EDGES = 320000
D_NODE = 128
D_EDGE = 16

NC = 2
NS = 16
NW = NC * NS
CH = 128
NCH = 80
PER_W = CH * NCH
E_PAD = NW * PER_W
N_HALF = 5
NCH_C = NCH // N_HALF
PER_WC = CH * NCH_C
E_C = NW * PER_WC
PAD_NODE = N_NODES
NODE_PAD = 10240
ROWS_PER_SUB = NODE_PAD // NS
ZCH = ROWS_PER_SUB // CH

_SQRT_HALF = 0.7071067811865476


def _gelu(x):
    return 0.5 * x * (1.0 + lax.erf(x * _SQRT_HALF))


def _sc_gather_body(table_hbm, sidx_hbm, didx_hbm, srows_hbm, drows_hbm,
                    sidx_v, didx_v, bs0, bs1, bd0, bd1,
                    gs0, gs1, gd0, gd1, ws0, ws1, wd0, wd1):
    c = lax.axis_index("c")
    s = lax.axis_index("s")
    wid = s * NC + c
    pltpu.sync_copy(sidx_hbm.at[pl.ds(wid * NCH_C, NCH_C)], sidx_v)
    pltpu.sync_copy(didx_hbm.at[pl.ds(wid * NCH_C, NCH_C)], didx_v)
    bs, bd = (bs0, bs1), (bd0, bd1)
    gs, gd = (gs0, gs1), (gd0, gd1)
    ws, wd = (ws0, ws1), (wd0, wd1)

    def start_gather(j, p):
        pltpu.async_copy(table_hbm.at[sidx_v.at[j]], bs[p], gs[p])
        pltpu.async_copy(table_hbm.at[didx_v.at[j]], bd[p], gd[p])

    def wait_writes(j, p):
        base = wid * PER_WC + j * CH
        pltpu.make_async_copy(bs[p], srows_hbm.at[pl.ds(base, CH)],
                              ws[p]).wait()
        pltpu.make_async_copy(bd[p], drows_hbm.at[pl.ds(base, CH)],
                              wd[p]).wait()

    start_gather(0, 0)

    def body(ii, carry):
        for p in (0, 1):
            j = 2 * ii + p
            np_ = 1 - p

            @pl.when(j + 1 < NCH_C)
            def _prefetch():
                @pl.when(j >= 1)
                def _drain():
                    wait_writes(j - 1, np_)
                start_gather(j + 1, np_)

            base = wid * PER_WC + j * CH
            pltpu.make_async_copy(table_hbm.at[sidx_v.at[j]], bs[p],
                                  gs[p]).wait()
            pltpu.async_copy(bs[p], srows_hbm.at[pl.ds(base, CH)], ws[p])
            pltpu.make_async_copy(table_hbm.at[didx_v.at[j]], bd[p],
                                  gd[p]).wait()
            pltpu.async_copy(bd[p], drows_hbm.at[pl.ds(base, CH)], wd[p])
        return carry

    lax.fori_loop(0, NCH_C // 2, body, 0)
    wait_writes(NCH_C - 2, 0)
    wait_writes(NCH_C - 1, 1)


@functools.cache
def _get_sc_gather():
    return pl.kernel(
        _sc_gather_body,
        out_type=(
            jax.ShapeDtypeStruct((E_C, D_NODE), jnp.float32),
            jax.ShapeDtypeStruct((E_C, D_NODE), jnp.float32),
        ),
        mesh=plsc.VectorSubcoreMesh(core_axis_name="c", subcore_axis_name="s",
                                    num_cores=NC, num_subcores=NS),
        scratch_types=(
            [pltpu.VMEM((NCH_C, CH), jnp.int32)] * 2
            + [pltpu.VMEM((CH, D_NODE), jnp.float32)] * 4
            + [pltpu.SemaphoreType.DMA] * 8
        ),
    )


def _tc_msg_body(srows, drows, ev, w1a, w1b, w1c, b1, w2, b2, msg1, msg2):
    sr = srows[...].astype(jnp.bfloat16)
    dr = drows[...].astype(jnp.bfloat16)
    w1a_ = w1a[...].astype(jnp.bfloat16)
    w1b_ = w1b[...].astype(jnp.bfloat16)
    w2_ = w2[...].astype(jnp.bfloat16)
    e = jnp.dot(ev[...], w1c[...], preferred_element_type=jnp.float32) + b1[...]
    sa = jnp.dot(sr, w1a_, preferred_element_type=jnp.float32)
    sb = jnp.dot(sr, w1b_, preferred_element_type=jnp.float32)
    da = jnp.dot(dr, w1a_, preferred_element_type=jnp.float32)
    db = jnp.dot(dr, w1b_, preferred_element_type=jnp.float32)
    h1 = _gelu(sa + db + e).astype(jnp.bfloat16)
    h2 = _gelu(da + sb + e).astype(jnp.bfloat16)
    msg1[...] = _gelu(jnp.dot(h1, w2_, preferred_element_type=jnp.float32)
                      + b2[...])
    msg2[...] = _gelu(jnp.dot(h2, w2_, preferred_element_type=jnp.float32)
                      + b2[...])


def _tc_msgs(srows, drows, ev, W1, b1, W2, b2, blk=4096):
    nblk = E_C // blk
    w1a = W1[:D_NODE]
    w1b = W1[D_NODE:2 * D_NODE]
    w1c = W1[2 * D_NODE:]
    d_hid = W1.shape[1]
    full = lambda a: pl.BlockSpec(a.shape, lambda i: (0,) * a.ndim)
    return pl.pallas_call(
        _tc_msg_body,
        grid=(nblk,),
        in_specs=[
            pl.BlockSpec((blk, D_NODE), lambda i: (i, 0)),
            pl.BlockSpec((blk, D_NODE), lambda i: (i, 0)),
            pl.BlockSpec((blk, D_EDGE), lambda i: (i, 0)),
            full(w1a), full(w1b), full(w1c),
            pl.BlockSpec((1, d_hid), lambda i: (0, 0)),
            full(W2),
            pl.BlockSpec((1, D_NODE), lambda i: (0, 0)),
        ],
        out_specs=[
            pl.BlockSpec((blk, D_NODE), lambda i: (i, 0)),
            pl.BlockSpec((blk, D_NODE), lambda i: (i, 0)),
        ],
        out_shape=[
            jax.ShapeDtypeStruct((E_C, D_NODE), jnp.float32),
            jax.ShapeDtypeStruct((E_C, D_NODE), jnp.float32),
        ],
        compiler_params=pltpu.CompilerParams(
            dimension_semantics=("arbitrary",)),
    )(srows, drows, ev, w1a, w1b, w1c, b1.reshape(1, -1), W2,
      b2.reshape(1, -1))


def _sc_scatter_body(msg1_hbm, msg2_hbm, sidx_hbm, didx_hbm, zeros_hbm,
                     out_hbm, sidx_v, didx_v, mba, mbb, boxes_sh, rsa, rsb):
    c = lax.axis_index("c")
    s = lax.axis_index("s")
    wid = s * NC + c

    pltpu.sync_copy(zeros_hbm, mba)

    def zero_chunk(k, carry):
        pltpu.sync_copy(mba, boxes_sh.at[pl.ds(s * ROWS_PER_SUB + k * CH, CH)])
        return carry

    lax.fori_loop(0, ZCH, zero_chunk, 0)
    plsc.subcore_barrier()

    pltpu.sync_copy(sidx_hbm.at[pl.ds(wid * NCH_C, NCH_C)], sidx_v)
    pltpu.sync_copy(didx_hbm.at[pl.ds(wid * NCH_C, NCH_C)], didx_v)

    base0 = wid * PER_WC
    pltpu.async_copy(msg1_hbm.at[pl.ds(base0, CH)], mba, rsa)

    def chunk(j, carry):
        base = base0 + j * CH
        pltpu.async_copy(msg2_hbm.at[pl.ds(base, CH)], mbb, rsb)
        pltpu.make_async_copy(msg1_hbm.at[pl.ds(base, CH)], mba, rsa).wait()
        pltpu.sync_copy(mba, boxes_sh.at[sidx_v.at[j]], add=True)

        @pl.when(j + 1 < NCH_C)
        def _prefetch():
            pltpu.async_copy(msg1_hbm.at[pl.ds(base + CH, CH)], mba, rsa)

        pltpu.make_async_copy(msg2_hbm.at[pl.ds(base, CH)], mbb, rsb).wait()
        pltpu.sync_copy(mbb, boxes_sh.at[didx_v.at[j]], add=True)
        return carry

    lax.fori_loop(0, NCH_C, chunk, 0)
    plsc.subcore_barrier()

    def dump_chunk(k, carry):
        r = s * ROWS_PER_SUB + k * CH
        pltpu.sync_copy(boxes_sh.at[pl.ds(r, CH)], mba)
        pltpu.sync_copy(mba, out_hbm.at[c, pl.ds(r, CH)])
        return carry

    lax.fori_loop(0, ZCH, dump_chunk, 0)


@functools.cache
def _get_sc_scatter():
    return pl.kernel(
        _sc_scatter_body,
        out_type=jax.ShapeDtypeStruct((NC, NODE_PAD, D_NODE), jnp.float32),
        mesh=plsc.VectorSubcoreMesh(core_axis_name="c", subcore_axis_name="s",
                                    num_cores=NC, num_subcores=NS),
        scratch_types=(
            [pltpu.VMEM((NCH_C, CH), jnp.int32)] * 2
            + [pltpu.VMEM((CH, D_NODE), jnp.float32)] * 2
            + [pltpu.VMEM_SHARED((NODE_PAD, D_NODE), jnp.float32)]
            + [pltpu.SemaphoreType.DMA] * 2
        ),
    )


def _tc_combine(nv, boxes, W3, b3, W4, b4, blk=2000):
    nblk = N_NODES // blk
    nb = len(boxes)
    w3a = W3[:D_NODE]
    w3b = W3[D_NODE:]
    d_hid = W3.shape[1]

    def body(*refs):
        nv_r = refs[0]
        box_rs = refs[1:1 + nb]
        w3a_r, w3b_r, b3_r, w4_r, b4_r, out = refs[1 + nb:]
        acc = box_rs[0][...]
        for r in box_rs[1:]:
            acc = acc + r[...]
        h = _gelu(
            jnp.dot(nv_r[...], w3a_r[...], preferred_element_type=jnp.float32)
            + jnp.dot(acc, w3b_r[...], preferred_element_type=jnp.float32)
            + b3_r[...])
        out[...] = _gelu(
            jnp.dot(h, w4_r[...], preferred_element_type=jnp.float32)
            + b4_r[...])

    full = lambda a: pl.BlockSpec(a.shape, lambda i: (0,) * a.ndim)
    row_spec = pl.BlockSpec((blk, D_NODE), lambda i: (i, 0))
    return pl.pallas_call(
        body,
        grid=(nblk,),
        in_specs=[row_spec] * (1 + nb) + [
            full(w3a), full(w3b),
            pl.BlockSpec((1, d_hid), lambda i: (0, 0)),
            full(W4),
            pl.BlockSpec((1, D_NODE), lambda i: (0, 0)),
        ],
        out_specs=pl.BlockSpec((blk, D_NODE), lambda i: (i, 0)),
        out_shape=jax.ShapeDtypeStruct((N_NODES, D_NODE), jnp.float32),
        compiler_params=pltpu.CompilerParams(
            dimension_semantics=("arbitrary",)),
    )(nv, *boxes, w3a, w3b, b3.reshape(1, -1), W4, b4.reshape(1, -1))


def kernel(node_vectors, edge_vectors, edge_indices, W1, b1, W2, b2, W3, b3,
           W4, b4):
    table = jnp.zeros((NODE_PAD, D_NODE), jnp.float32)
    table = table.at[:N_NODES].set(node_vectors)
    npad = E_PAD - N_EDGES
    src = jnp.concatenate(
        [edge_indices[:, 0], jnp.full((npad,), PAD_NODE, jnp.int32)])
    dst = jnp.concatenate(
        [edge_indices[:, 1], jnp.full((npad,), PAD_NODE, jnp.int32)])
    sidx = src.reshape(NW * NCH, CH)
    didx = dst.reshape(NW * NCH, CH)
    ev = jnp.concatenate(
        [edge_vectors, jnp.zeros((npad, D_EDGE), jnp.float32)])
    zeros_blk = jnp.zeros((CH, D_NODE), jnp.float32)

    gather = _get_sc_gather()
    scatter = _get_sc_scatter()
    boxes4 = []
    for h in range(N_HALF):
        rows = slice(h * NW * NCH_C, (h + 1) * NW * NCH_C)
        erows = slice(h * E_C, (h + 1) * E_C)
        sidx_h, didx_h = sidx[rows], didx[rows]
        srows, drows = gather(table, sidx_h, didx_h)
        msg1, msg2 = _tc_msgs(srows, drows, ev[erows], W1, b1, W2, b2)
        partials = scatter(msg1, msg2, sidx_h, didx_h, zeros_blk)
        boxes4.extend([partials[0, :N_NODES], partials[1, :N_NODES]])
    return _tc_combine(node_vectors, boxes4, W3, b3, W4, b4)
